# Initial kernel scaffold; baseline (speedup 1.0000x reference)
import jax
import jax.numpy as jnp
from jax.experimental import pallas as pl


def kernel(x_user, x_item, edge_index_ui, edge_index_iu, batch_user, batch_item, W_l1_ui, b1_ui, W_r1_ui, W_l1_iu, b1_iu, W_r1_iu, W_l2_ui, b2_ui, W_r2_ui, W_l2_iu, b2_iu, W_r2_iu, lin_W, lin_b):
    raise NotImplementedError("write your pallas kernel here")


# speedup vs baseline: 4.2720x; 4.2720x over previous
import functools

import jax
import jax.numpy as jnp
from jax import lax
from jax.experimental import pallas as pl
from jax.experimental.pallas import tpu as pltpu
from jax.experimental.pallas import tpu_sc as plsc

N = 10000
E = 320000
D = 128
H = 64
O = 128
G = 64

NSUB = 16
CH = 128
CHUNKS = -(-E // CH)
CHUNKS_PER_TILE = 160
CHUNKS_PAD = CHUNKS_PER_TILE * NSUB
E_PAD = CHUNKS_PAD * CH
ROWS_PER_TILE = 632
N_PAD = ROWS_PER_TILE * NSUB
ZBLK = 128

_mesh = plsc.VectorSubcoreMesh(core_axis_name="c", subcore_axis_name="s")


def _sc_agg_body(with_counts, xw_hbm, src_hbm, dst_hbm, *refs):
    if with_counts:
        (agg_hbm, cnt_hbm, acc_sh, cnt_sh, src_v, dst_v, rows_v, ones_v,
         zrow_v, zcnt_v, sem) = refs
    else:
        agg_hbm, acc_sh, src_v, dst_v, rows_v, zrow_v, sem = refs

    cid = lax.axis_index("c")
    sid = lax.axis_index("s")
    zero16 = jnp.zeros((16,), jnp.float32)

    @pl.loop(0, ZBLK)
    def _(r):
        @pl.loop(0, H, step=16)
        def _(k):
            zrow_v[r, pl.ds(k, 16)] = zero16

    base = sid * ROWS_PER_TILE
    @pl.loop(0, 4)
    def _(b):
        pltpu.sync_copy(zrow_v, acc_sh.at[pl.ds(base + b * ZBLK, ZBLK)])
    pltpu.sync_copy(zrow_v.at[pl.ds(0, ROWS_PER_TILE - 4 * ZBLK)],
                    acc_sh.at[pl.ds(base + 4 * ZBLK, ROWS_PER_TILE - 4 * ZBLK)])

    if with_counts:
        one16 = jnp.ones((16,), jnp.float32)

        @pl.loop(0, ZBLK)
        def _(r):
            ones_v[r, pl.ds(0, 16)] = one16
            zcnt_v[r, pl.ds(0, 16)] = zero16

        @pl.loop(0, 4)
        def _(b):
            pltpu.sync_copy(zcnt_v, cnt_sh.at[pl.ds(base + b * ZBLK, ZBLK)])
        pltpu.sync_copy(zcnt_v.at[pl.ds(0, ROWS_PER_TILE - 4 * ZBLK)],
                        cnt_sh.at[pl.ds(base + 4 * ZBLK, ROWS_PER_TILE - 4 * ZBLK)])

    plsc.subcore_barrier()

    @pl.loop(0, CHUNKS_PER_TILE // 8)
    def _(grp):
        g = sid * CHUNKS_PER_TILE + grp * 8
        pltpu.sync_copy(src_hbm.at[cid, pl.ds(g, 8)], src_v)
        pltpu.sync_copy(dst_hbm.at[cid, pl.ds(g, 8)], dst_v)

        @pl.loop(0, 8)
        def _(r):
            pltpu.async_copy(xw_hbm.at[src_v.at[r]], rows_v, sem).wait()
            pltpu.sync_copy(rows_v, acc_sh.at[dst_v.at[r]], add=True)
            if with_counts:
                pltpu.sync_copy(ones_v, cnt_sh.at[dst_v.at[r]], add=True)

    plsc.subcore_barrier()

    pltpu.sync_copy(acc_sh.at[pl.ds(base, ROWS_PER_TILE)],
                    agg_hbm.at[cid, pl.ds(base, ROWS_PER_TILE)])
    if with_counts:
        pltpu.sync_copy(cnt_sh.at[pl.ds(base, ROWS_PER_TILE)],
                        cnt_hbm.at[cid, pl.ds(base, ROWS_PER_TILE)])


def _make_sc_agg(with_counts):
    out_type = [jax.ShapeDtypeStruct((2, N_PAD, H), jnp.float32)]
    scratch = [
        pltpu.VMEM_SHARED((N_PAD, H), jnp.float32),
    ]
    if with_counts:
        out_type.append(jax.ShapeDtypeStruct((2, N_PAD, 16), jnp.float32))
        scratch.append(pltpu.VMEM_SHARED((N_PAD, 16), jnp.float32))
    scratch += [
        pltpu.VMEM((8, CH), jnp.int32),
        pltpu.VMEM((8, CH), jnp.int32),
        pltpu.VMEM((CH, H), jnp.float32),
    ]
    if with_counts:
        scratch.append(pltpu.VMEM((CH, 16), jnp.float32))
    scratch.append(pltpu.VMEM((ZBLK, H), jnp.float32))
    if with_counts:
        scratch.append(pltpu.VMEM((ZBLK, 16), jnp.float32))
    scratch.append(pltpu.SemaphoreType.DMA)
    return pl.kernel(
        functools.partial(_sc_agg_body, with_counts),
        out_type=tuple(out_type) if with_counts else out_type[0],
        mesh=_mesh,
        scratch_types=scratch,
        compiler_params=pltpu.CompilerParams(use_tc_tiling_on_sc=False),
    )


_sc_agg_counts = _make_sc_agg(True)
_sc_agg_plain = _make_sc_agg(False)


def _dotT(x, w):
    return lax.dot_general(x, w, (((1,), (1,)), ((), ())),
                           preferred_element_type=jnp.float32)


def _pre1_body(xu, xi, wui, wiu, out):
    out[pl.ds(0, N), :] = _dotT(xu[...], wui[...])
    out[pl.ds(N, N), :] = _dotT(xi[...], wiu[...])


def _tc_pre1(x_user, x_item, wl_ui, wl_iu):
    return pl.pallas_call(
        _pre1_body,
        out_shape=jax.ShapeDtypeStruct((2 * N, H), jnp.float32),
    )(x_user, x_item, wl_ui, wl_iu)


def _stageb_body(agg, cnt, xu, xi, wr_ui, wr_iu, b_ui, b_iu, wl2_ui, wl2_iu,
                 item1_o, user1_o, xw2_o):
    cnt_ui = jnp.maximum(cnt[0, pl.ds(0, N), pl.ds(0, 1)], 1.0)
    cnt_iu = jnp.maximum(cnt[1, pl.ds(0, N), pl.ds(0, 1)], 1.0)
    item1 = jax.nn.relu(agg[0, pl.ds(0, N), :] / cnt_ui + b_ui[...]
                        + _dotT(xi[...], wr_ui[...]))
    user1 = jax.nn.relu(agg[1, pl.ds(0, N), :] / cnt_iu + b_iu[...]
                        + _dotT(xu[...], wr_iu[...]))
    item1_o[...] = item1
    user1_o[...] = user1
    xw2_o[pl.ds(0, N), :] = _dotT(user1, wl2_ui[...])
    xw2_o[pl.ds(N, N), :] = _dotT(item1, wl2_iu[...])


def _tc_stageb(agg, cnt, x_user, x_item, wr_ui, wr_iu, b_ui, b_iu,
               wl2_ui, wl2_iu):
    return pl.pallas_call(
        _stageb_body,
        out_shape=(
            jax.ShapeDtypeStruct((N, H), jnp.float32),
            jax.ShapeDtypeStruct((N, H), jnp.float32),
            jax.ShapeDtypeStruct((2 * N, H), jnp.float32),
        ),
    )(agg, cnt, x_user, x_item, wr_ui, wr_iu, b_ui, b_iu, wl2_ui, wl2_iu)


def _stagec_body(agg, cnt, item1, user1, wr_ui, wr_iu, b_ui, b_iu,
                 batch_u, batch_i, lin_w, lin_b, out):
    cnt_ui = jnp.maximum(cnt[0, pl.ds(0, N), pl.ds(0, 1)], 1.0)
    cnt_iu = jnp.maximum(cnt[1, pl.ds(0, N), pl.ds(0, 1)], 1.0)
    item2 = jax.nn.relu(agg[0, pl.ds(0, N), :] / cnt_ui + b_ui[...]
                        + _dotT(item1[...], wr_ui[...]))
    user2 = jax.nn.relu(agg[1, pl.ds(0, N), :] / cnt_iu + b_iu[...]
                        + _dotT(user1[...], wr_iu[...]))
    gids = lax.broadcasted_iota(jnp.int32, (1, G), 1)
    oh_u = (batch_u[...] == gids).astype(jnp.float32)
    oh_i = (batch_i[...] == gids).astype(jnp.float32)
    pool_dims = (((0,), (0,)), ((), ()))
    pu = lax.dot_general(oh_u, user2, pool_dims,
                         preferred_element_type=jnp.float32)
    pi = lax.dot_general(oh_i, item2, pool_dims,
                         preferred_element_type=jnp.float32)
    cu = jnp.maximum(jnp.sum(oh_u, axis=0, keepdims=True), 1.0)
    ci = jnp.maximum(jnp.sum(oh_i, axis=0, keepdims=True), 1.0)
    g = pu / cu.T + pi / ci.T
    out[...] = _dotT(g, lin_w[...]) + lin_b[...]


def _tc_stagec(agg, cnt, item1, user1, wr_ui, wr_iu, b_ui, b_iu,
               batch_u, batch_i, lin_w, lin_b):
    return pl.pallas_call(
        _stagec_body,
        out_shape=jax.ShapeDtypeStruct((G, O), jnp.float32),
    )(agg, cnt, item1, user1, wr_ui, wr_iu, b_ui, b_iu,
      batch_u, batch_i, lin_w, lin_b)


def kernel(x_user, x_item, edge_index_ui, edge_index_iu, batch_user,
           batch_item, W_l1_ui, b1_ui, W_r1_ui, W_l1_iu, b1_iu, W_r1_iu,
           W_l2_ui, b2_ui, W_r2_ui, W_l2_iu, b2_iu, W_r2_iu, lin_W, lin_b):
    pad = E_PAD - E
    src_all = jnp.stack([
        jnp.concatenate([edge_index_ui[0], jnp.zeros((pad,), jnp.int32)]),
        jnp.concatenate([edge_index_iu[0] + N, jnp.zeros((pad,), jnp.int32)]),
    ]).reshape(2, CHUNKS_PAD, CH)
    dump = jnp.full((pad,), N, jnp.int32)
    dst_all = jnp.stack([
        jnp.concatenate([edge_index_ui[1], dump]),
        jnp.concatenate([edge_index_iu[1], dump]),
    ]).reshape(2, CHUNKS_PAD, CH)

    xw1 = _tc_pre1(x_user, x_item, W_l1_ui, W_l1_iu)
    agg1, cnt = _sc_agg_counts(xw1, src_all, dst_all)
    item1, user1, xw2 = _tc_stageb(agg1, cnt, x_user, x_item, W_r1_ui,
                                   W_r1_iu, b1_ui, b1_iu, W_l2_ui, W_l2_iu)
    agg2 = _sc_agg_plain(xw2, src_all, dst_all)
    return _tc_stagec(agg2, cnt, item1, user1, W_r2_ui, W_r2_iu, b2_ui,
                      b2_iu, batch_user.reshape(N, 1), batch_item.reshape(N, 1),
                      lin_W, lin_b)


# speedup vs baseline: 5.0457x; 1.1811x over previous
import functools

import jax
import jax.numpy as jnp
from jax import lax
from jax.experimental import pallas as pl
from jax.experimental.pallas import tpu as pltpu
from jax.experimental.pallas import tpu_sc as plsc

N = 10000
E = 320000
D = 128
H = 64
O = 128
G = 64

NSUB = 16
CH = 128
CHUNKS = -(-E // CH)
CHUNKS_PER_TILE = 160
CHUNKS_PAD = CHUNKS_PER_TILE * NSUB
E_PAD = CHUNKS_PAD * CH
ROWS_PER_TILE = 632
N_PAD = ROWS_PER_TILE * NSUB
ZBLK = 128

_mesh = plsc.VectorSubcoreMesh(core_axis_name="c", subcore_axis_name="s")


def _sc_agg_body(with_counts, xw_hbm, src_hbm, dst_hbm, *refs):
    if with_counts:
        (agg_hbm, cnt_hbm, acc_sh, cnt_sh, src_v, dst_v, rows0_v, rows1_v,
         ones_v, zrow_v, zcnt_v, sem0, sem1) = refs
    else:
        (agg_hbm, acc_sh, src_v, dst_v, rows0_v, rows1_v, zrow_v,
         sem0, sem1) = refs

    cid = lax.axis_index("c")
    sid = lax.axis_index("s")
    zero16 = jnp.zeros((16,), jnp.float32)

    @pl.loop(0, ZBLK)
    def _(r):
        @pl.loop(0, H, step=16)
        def _(k):
            zrow_v[r, pl.ds(k, 16)] = zero16

    base = sid * ROWS_PER_TILE
    @pl.loop(0, 4)
    def _(b):
        pltpu.sync_copy(zrow_v, acc_sh.at[pl.ds(base + b * ZBLK, ZBLK)])
    pltpu.sync_copy(zrow_v.at[pl.ds(0, ROWS_PER_TILE - 4 * ZBLK)],
                    acc_sh.at[pl.ds(base + 4 * ZBLK, ROWS_PER_TILE - 4 * ZBLK)])

    if with_counts:
        one16 = jnp.ones((16,), jnp.float32)

        @pl.loop(0, ZBLK)
        def _(r):
            ones_v[r, pl.ds(0, 16)] = one16
            zcnt_v[r, pl.ds(0, 16)] = zero16

        @pl.loop(0, 4)
        def _(b):
            pltpu.sync_copy(zcnt_v, cnt_sh.at[pl.ds(base + b * ZBLK, ZBLK)])
        pltpu.sync_copy(zcnt_v.at[pl.ds(0, ROWS_PER_TILE - 4 * ZBLK)],
                        cnt_sh.at[pl.ds(base + 4 * ZBLK, ROWS_PER_TILE - 4 * ZBLK)])

    plsc.subcore_barrier()

    tb = sid * CHUNKS_PER_TILE
    pltpu.sync_copy(src_hbm.at[cid, pl.ds(tb, CHUNKS_PER_TILE)], src_v)
    pltpu.sync_copy(dst_hbm.at[cid, pl.ds(tb, CHUNKS_PER_TILE)], dst_v)

    def gstart(j, buf, sem):
        pltpu.async_copy(xw_hbm.at[src_v.at[j]], buf, sem)

    def gwait(j, buf, sem):
        pltpu.make_async_copy(xw_hbm.at[src_v.at[j]], buf, sem).wait()

    def scat(j, buf):
        pltpu.sync_copy(buf, acc_sh.at[dst_v.at[j]], add=True)
        if with_counts:
            pltpu.sync_copy(ones_v, cnt_sh.at[dst_v.at[j]], add=True)

    gstart(0, rows0_v, sem0)

    @pl.loop(0, CHUNKS_PER_TILE, step=2)
    def _(j):
        gstart(j + 1, rows1_v, sem1)
        gwait(j, rows0_v, sem0)
        scat(j, rows0_v)

        @pl.when(j + 2 < CHUNKS_PER_TILE)
        def _():
            gstart(j + 2, rows0_v, sem0)

        gwait(j + 1, rows1_v, sem1)
        scat(j + 1, rows1_v)

    plsc.subcore_barrier()

    pltpu.sync_copy(acc_sh.at[pl.ds(base, ROWS_PER_TILE)],
                    agg_hbm.at[cid, pl.ds(base, ROWS_PER_TILE)])
    if with_counts:
        pltpu.sync_copy(cnt_sh.at[pl.ds(base, ROWS_PER_TILE)],
                        cnt_hbm.at[cid, pl.ds(base, ROWS_PER_TILE)])


def _make_sc_agg(with_counts):
    out_type = [jax.ShapeDtypeStruct((2, N_PAD, H), jnp.float32)]
    scratch = [
        pltpu.VMEM_SHARED((N_PAD, H), jnp.float32),
    ]
    if with_counts:
        out_type.append(jax.ShapeDtypeStruct((2, N_PAD, 16), jnp.float32))
        scratch.append(pltpu.VMEM_SHARED((N_PAD, 16), jnp.float32))
    scratch += [
        pltpu.VMEM((CHUNKS_PER_TILE, CH), jnp.int32),
        pltpu.VMEM((CHUNKS_PER_TILE, CH), jnp.int32),
        pltpu.VMEM((CH, H), jnp.float32),
        pltpu.VMEM((CH, H), jnp.float32),
    ]
    if with_counts:
        scratch.append(pltpu.VMEM((CH, 16), jnp.float32))
    scratch.append(pltpu.VMEM((ZBLK, H), jnp.float32))
    if with_counts:
        scratch.append(pltpu.VMEM((ZBLK, 16), jnp.float32))
    scratch += [pltpu.SemaphoreType.DMA, pltpu.SemaphoreType.DMA]
    return pl.kernel(
        functools.partial(_sc_agg_body, with_counts),
        out_type=tuple(out_type) if with_counts else out_type[0],
        mesh=_mesh,
        scratch_types=scratch,
        compiler_params=pltpu.CompilerParams(use_tc_tiling_on_sc=False),
    )


_sc_agg_counts = _make_sc_agg(True)
_sc_agg_plain = _make_sc_agg(False)


def _dotT(x, w):
    return lax.dot_general(x, w, (((1,), (1,)), ((), ())),
                           preferred_element_type=jnp.float32)


def _pre1_body(xu, xi, wui, wiu, out):
    out[pl.ds(0, N), :] = _dotT(xu[...], wui[...])
    out[pl.ds(N, N), :] = _dotT(xi[...], wiu[...])


def _tc_pre1(x_user, x_item, wl_ui, wl_iu):
    return pl.pallas_call(
        _pre1_body,
        out_shape=jax.ShapeDtypeStruct((2 * N, H), jnp.float32),
    )(x_user, x_item, wl_ui, wl_iu)


def _stageb_body(agg, cnt, xu, xi, wr_ui, wr_iu, b_ui, b_iu, wl2_ui, wl2_iu,
                 item1_o, user1_o, xw2_o):
    cnt_ui = jnp.maximum(cnt[0, pl.ds(0, N), pl.ds(0, 1)], 1.0)
    cnt_iu = jnp.maximum(cnt[1, pl.ds(0, N), pl.ds(0, 1)], 1.0)
    item1 = jax.nn.relu(agg[0, pl.ds(0, N), :] / cnt_ui + b_ui[...]
                        + _dotT(xi[...], wr_ui[...]))
    user1 = jax.nn.relu(agg[1, pl.ds(0, N), :] / cnt_iu + b_iu[...]
                        + _dotT(xu[...], wr_iu[...]))
    item1_o[...] = item1
    user1_o[...] = user1
    xw2_o[pl.ds(0, N), :] = _dotT(user1, wl2_ui[...])
    xw2_o[pl.ds(N, N), :] = _dotT(item1, wl2_iu[...])


def _tc_stageb(agg, cnt, x_user, x_item, wr_ui, wr_iu, b_ui, b_iu,
               wl2_ui, wl2_iu):
    return pl.pallas_call(
        _stageb_body,
        out_shape=(
            jax.ShapeDtypeStruct((N, H), jnp.float32),
            jax.ShapeDtypeStruct((N, H), jnp.float32),
            jax.ShapeDtypeStruct((2 * N, H), jnp.float32),
        ),
    )(agg, cnt, x_user, x_item, wr_ui, wr_iu, b_ui, b_iu, wl2_ui, wl2_iu)


def _stagec_body(agg, cnt, item1, user1, wr_ui, wr_iu, b_ui, b_iu,
                 batch_u, batch_i, lin_w, lin_b, out):
    cnt_ui = jnp.maximum(cnt[0, pl.ds(0, N), pl.ds(0, 1)], 1.0)
    cnt_iu = jnp.maximum(cnt[1, pl.ds(0, N), pl.ds(0, 1)], 1.0)
    item2 = jax.nn.relu(agg[0, pl.ds(0, N), :] / cnt_ui + b_ui[...]
                        + _dotT(item1[...], wr_ui[...]))
    user2 = jax.nn.relu(agg[1, pl.ds(0, N), :] / cnt_iu + b_iu[...]
                        + _dotT(user1[...], wr_iu[...]))
    gids = lax.broadcasted_iota(jnp.int32, (1, G), 1)
    oh_u = (batch_u[...] == gids).astype(jnp.float32)
    oh_i = (batch_i[...] == gids).astype(jnp.float32)
    pool_dims = (((0,), (0,)), ((), ()))
    pu = lax.dot_general(oh_u, user2, pool_dims,
                         preferred_element_type=jnp.float32)
    pi = lax.dot_general(oh_i, item2, pool_dims,
                         preferred_element_type=jnp.float32)
    cu = jnp.maximum(jnp.sum(oh_u, axis=0, keepdims=True), 1.0)
    ci = jnp.maximum(jnp.sum(oh_i, axis=0, keepdims=True), 1.0)
    g = pu / cu.T + pi / ci.T
    out[...] = _dotT(g, lin_w[...]) + lin_b[...]


def _tc_stagec(agg, cnt, item1, user1, wr_ui, wr_iu, b_ui, b_iu,
               batch_u, batch_i, lin_w, lin_b):
    return pl.pallas_call(
        _stagec_body,
        out_shape=jax.ShapeDtypeStruct((G, O), jnp.float32),
    )(agg, cnt, item1, user1, wr_ui, wr_iu, b_ui, b_iu,
      batch_u, batch_i, lin_w, lin_b)


def kernel(x_user, x_item, edge_index_ui, edge_index_iu, batch_user,
           batch_item, W_l1_ui, b1_ui, W_r1_ui, W_l1_iu, b1_iu, W_r1_iu,
           W_l2_ui, b2_ui, W_r2_ui, W_l2_iu, b2_iu, W_r2_iu, lin_W, lin_b):
    pad = E_PAD - E
    src_all = jnp.stack([
        jnp.concatenate([edge_index_ui[0], jnp.zeros((pad,), jnp.int32)]),
        jnp.concatenate([edge_index_iu[0] + N, jnp.zeros((pad,), jnp.int32)]),
    ]).reshape(2, CHUNKS_PAD, CH)
    dump = jnp.full((pad,), N, jnp.int32)
    dst_all = jnp.stack([
        jnp.concatenate([edge_index_ui[1], dump]),
        jnp.concatenate([edge_index_iu[1], dump]),
    ]).reshape(2, CHUNKS_PAD, CH)

    xw1 = _tc_pre1(x_user, x_item, W_l1_ui, W_l1_iu)
    agg1, cnt = _sc_agg_counts(xw1, src_all, dst_all)
    item1, user1, xw2 = _tc_stageb(agg1, cnt, x_user, x_item, W_r1_ui,
                                   W_r1_iu, b1_ui, b1_iu, W_l2_ui, W_l2_iu)
    agg2 = _sc_agg_plain(xw2, src_all, dst_all)
    return _tc_stagec(agg2, cnt, item1, user1, W_r2_ui, W_r2_iu, b2_ui,
                      b2_iu, batch_user.reshape(N, 1), batch_item.reshape(N, 1),
                      lin_W, lin_b)


# speedup vs baseline: 9.4751x; 1.8778x over previous
import functools

import jax
import jax.numpy as jnp
from jax import lax
from jax.experimental import pallas as pl
from jax.experimental.pallas import tpu as pltpu
from jax.experimental.pallas import tpu_sc as plsc

N = 10000
E = 320000
D = 128
H = 64
O = 128
G = 64

NSUB = 16
CH = 128
CHUNKS = -(-E // CH)
CHUNKS_PER_TILE = 160
CHUNKS_PAD = CHUNKS_PER_TILE * NSUB
E_PAD = CHUNKS_PAD * CH
ROWS_PER_TILE = 632
N_PAD = ROWS_PER_TILE * NSUB
ZBLK = 128
TN = N
TROWS = TN // NSUB
IDXB = 32

_mesh = plsc.VectorSubcoreMesh(core_axis_name="c", subcore_axis_name="s")


def _sc_agg_body(with_counts, xw_hbm, src_hbm, dst_hbm, *refs):
    if with_counts:
        (agg_hbm, cnt_hbm, acc_sh, cnt_sh, xw_sh, src_v, dst_v, rows0_v,
         rows1_v, ones_v, zcnt_v, sem0, sem1) = refs
    else:
        (agg_hbm, acc_sh, xw_sh, src_v, dst_v, rows0_v, rows1_v,
         sem0, sem1) = refs

    cid = lax.axis_index("c")
    sid = lax.axis_index("s")
    zero16 = jnp.zeros((16,), jnp.float32)

    @pl.loop(0, ZBLK)
    def _(r):
        @pl.loop(0, H, step=16)
        def _(k):
            rows0_v[r, pl.ds(k, 16)] = zero16

    base = sid * ROWS_PER_TILE
    @pl.loop(0, 4)
    def _(b):
        pltpu.sync_copy(rows0_v, acc_sh.at[pl.ds(base + b * ZBLK, ZBLK)])
    pltpu.sync_copy(rows0_v.at[pl.ds(0, ROWS_PER_TILE - 4 * ZBLK)],
                    acc_sh.at[pl.ds(base + 4 * ZBLK, ROWS_PER_TILE - 4 * ZBLK)])

    if with_counts:
        one16 = jnp.ones((16,), jnp.float32)

        @pl.loop(0, ZBLK)
        def _(r):
            ones_v[r, pl.ds(0, 16)] = one16
            zcnt_v[r, pl.ds(0, 16)] = zero16

        @pl.loop(0, 4)
        def _(b):
            pltpu.sync_copy(zcnt_v, cnt_sh.at[pl.ds(base + b * ZBLK, ZBLK)])
        pltpu.sync_copy(zcnt_v.at[pl.ds(0, ROWS_PER_TILE - 4 * ZBLK)],
                        cnt_sh.at[pl.ds(base + 4 * ZBLK, ROWS_PER_TILE - 4 * ZBLK)])

    pltpu.sync_copy(xw_hbm.at[pl.ds(cid * TN + sid * TROWS, TROWS)],
                    xw_sh.at[pl.ds(sid * TROWS, TROWS)])
    plsc.subcore_barrier()

    def gstart(j, buf, sem):
        pltpu.async_copy(xw_sh.at[src_v.at[j]], buf, sem)

    def gwait(j, buf, sem):
        pltpu.make_async_copy(xw_sh.at[src_v.at[j]], buf, sem).wait()

    def scat(j, buf):
        pltpu.sync_copy(buf, acc_sh.at[dst_v.at[j]], add=True)
        if with_counts:
            pltpu.sync_copy(ones_v, cnt_sh.at[dst_v.at[j]], add=True)

    tb = sid * CHUNKS_PER_TILE

    @pl.loop(0, CHUNKS_PER_TILE // IDXB)
    def _(b):
        blk = tb + b * IDXB
        pltpu.sync_copy(src_hbm.at[cid, pl.ds(blk, IDXB)], src_v)
        pltpu.sync_copy(dst_hbm.at[cid, pl.ds(blk, IDXB)], dst_v)
        gstart(0, rows0_v, sem0)

        @pl.loop(0, IDXB, step=2)
        def _(j):
            gstart(j + 1, rows1_v, sem1)
            gwait(j, rows0_v, sem0)
            scat(j, rows0_v)

            @pl.when(j + 2 < IDXB)
            def _():
                gstart(j + 2, rows0_v, sem0)

            gwait(j + 1, rows1_v, sem1)
            scat(j + 1, rows1_v)

    plsc.subcore_barrier()

    pltpu.sync_copy(acc_sh.at[pl.ds(base, ROWS_PER_TILE)],
                    agg_hbm.at[cid, pl.ds(base, ROWS_PER_TILE)])
    if with_counts:
        pltpu.sync_copy(cnt_sh.at[pl.ds(base, ROWS_PER_TILE)],
                        cnt_hbm.at[cid, pl.ds(base, ROWS_PER_TILE)])


def _make_sc_agg(with_counts):
    out_type = [jax.ShapeDtypeStruct((2, N_PAD, H), jnp.float32)]
    scratch = [
        pltpu.VMEM_SHARED((N_PAD, H), jnp.float32),
    ]
    if with_counts:
        out_type.append(jax.ShapeDtypeStruct((2, N_PAD, 16), jnp.float32))
        scratch.append(pltpu.VMEM_SHARED((N_PAD, 16), jnp.float32))
    scratch += [
        pltpu.VMEM_SHARED((TN, H), jnp.float32),
        pltpu.VMEM((IDXB, CH), jnp.int32),
        pltpu.VMEM((IDXB, CH), jnp.int32),
        pltpu.VMEM((CH, H), jnp.float32),
        pltpu.VMEM((CH, H), jnp.float32),
    ]
    if with_counts:
        scratch += [pltpu.VMEM((CH, 16), jnp.float32),
                    pltpu.VMEM((ZBLK, 16), jnp.float32)]
    scratch += [pltpu.SemaphoreType.DMA, pltpu.SemaphoreType.DMA]
    return pl.kernel(
        functools.partial(_sc_agg_body, with_counts),
        out_type=tuple(out_type) if with_counts else out_type[0],
        mesh=_mesh,
        scratch_types=scratch,
        compiler_params=pltpu.CompilerParams(use_tc_tiling_on_sc=False),
    )


_sc_agg_counts = _make_sc_agg(True)
_sc_agg_plain = _make_sc_agg(False)


def _dotT(x, w):
    return lax.dot_general(x, w, (((1,), (1,)), ((), ())),
                           preferred_element_type=jnp.float32)


def _pre1_body(xu, xi, wui, wiu, out):
    out[pl.ds(0, N), :] = _dotT(xu[...], wui[...])
    out[pl.ds(N, N), :] = _dotT(xi[...], wiu[...])


def _tc_pre1(x_user, x_item, wl_ui, wl_iu):
    return pl.pallas_call(
        _pre1_body,
        out_shape=jax.ShapeDtypeStruct((2 * N, H), jnp.float32),
    )(x_user, x_item, wl_ui, wl_iu)


def _stageb_body(agg, cnt, xu, xi, wr_ui, wr_iu, b_ui, b_iu, wl2_ui, wl2_iu,
                 item1_o, user1_o, xw2_o):
    cnt_ui = jnp.maximum(cnt[0, pl.ds(0, N), pl.ds(0, 1)], 1.0)
    cnt_iu = jnp.maximum(cnt[1, pl.ds(0, N), pl.ds(0, 1)], 1.0)
    item1 = jax.nn.relu(agg[0, pl.ds(0, N), :] / cnt_ui + b_ui[...]
                        + _dotT(xi[...], wr_ui[...]))
    user1 = jax.nn.relu(agg[1, pl.ds(0, N), :] / cnt_iu + b_iu[...]
                        + _dotT(xu[...], wr_iu[...]))
    item1_o[...] = item1
    user1_o[...] = user1
    xw2_o[pl.ds(0, N), :] = _dotT(user1, wl2_ui[...])
    xw2_o[pl.ds(N, N), :] = _dotT(item1, wl2_iu[...])


def _tc_stageb(agg, cnt, x_user, x_item, wr_ui, wr_iu, b_ui, b_iu,
               wl2_ui, wl2_iu):
    return pl.pallas_call(
        _stageb_body,
        out_shape=(
            jax.ShapeDtypeStruct((N, H), jnp.float32),
            jax.ShapeDtypeStruct((N, H), jnp.float32),
            jax.ShapeDtypeStruct((2 * N, H), jnp.float32),
        ),
    )(agg, cnt, x_user, x_item, wr_ui, wr_iu, b_ui, b_iu, wl2_ui, wl2_iu)


def _stagec_body(agg, cnt, item1, user1, wr_ui, wr_iu, b_ui, b_iu,
                 batch_u, batch_i, lin_w, lin_b, out):
    cnt_ui = jnp.maximum(cnt[0, pl.ds(0, N), pl.ds(0, 1)], 1.0)
    cnt_iu = jnp.maximum(cnt[1, pl.ds(0, N), pl.ds(0, 1)], 1.0)
    item2 = jax.nn.relu(agg[0, pl.ds(0, N), :] / cnt_ui + b_ui[...]
                        + _dotT(item1[...], wr_ui[...]))
    user2 = jax.nn.relu(agg[1, pl.ds(0, N), :] / cnt_iu + b_iu[...]
                        + _dotT(user1[...], wr_iu[...]))
    gids = lax.broadcasted_iota(jnp.int32, (1, G), 1)
    oh_u = (batch_u[...] == gids).astype(jnp.float32)
    oh_i = (batch_i[...] == gids).astype(jnp.float32)
    pool_dims = (((0,), (0,)), ((), ()))
    pu = lax.dot_general(oh_u, user2, pool_dims,
                         preferred_element_type=jnp.float32)
    pi = lax.dot_general(oh_i, item2, pool_dims,
                         preferred_element_type=jnp.float32)
    cu = jnp.maximum(jnp.sum(oh_u, axis=0, keepdims=True), 1.0)
    ci = jnp.maximum(jnp.sum(oh_i, axis=0, keepdims=True), 1.0)
    g = pu / cu.T + pi / ci.T
    out[...] = _dotT(g, lin_w[...]) + lin_b[...]


def _tc_stagec(agg, cnt, item1, user1, wr_ui, wr_iu, b_ui, b_iu,
               batch_u, batch_i, lin_w, lin_b):
    return pl.pallas_call(
        _stagec_body,
        out_shape=jax.ShapeDtypeStruct((G, O), jnp.float32),
    )(agg, cnt, item1, user1, wr_ui, wr_iu, b_ui, b_iu,
      batch_u, batch_i, lin_w, lin_b)


def kernel(x_user, x_item, edge_index_ui, edge_index_iu, batch_user,
           batch_item, W_l1_ui, b1_ui, W_r1_ui, W_l1_iu, b1_iu, W_r1_iu,
           W_l2_ui, b2_ui, W_r2_ui, W_l2_iu, b2_iu, W_r2_iu, lin_W, lin_b):
    pad = E_PAD - E
    src_all = jnp.stack([
        jnp.concatenate([edge_index_ui[0], jnp.zeros((pad,), jnp.int32)]),
        jnp.concatenate([edge_index_iu[0], jnp.zeros((pad,), jnp.int32)]),
    ]).reshape(2, CHUNKS_PAD, CH)
    dump = jnp.full((pad,), N, jnp.int32)
    dst_all = jnp.stack([
        jnp.concatenate([edge_index_ui[1], dump]),
        jnp.concatenate([edge_index_iu[1], dump]),
    ]).reshape(2, CHUNKS_PAD, CH)

    xw1 = _tc_pre1(x_user, x_item, W_l1_ui, W_l1_iu)
    agg1, cnt = _sc_agg_counts(xw1, src_all, dst_all)
    item1, user1, xw2 = _tc_stageb(agg1, cnt, x_user, x_item, W_r1_ui,
                                   W_r1_iu, b1_ui, b1_iu, W_l2_ui, W_l2_iu)
    agg2 = _sc_agg_plain(xw2, src_all, dst_all)
    return _tc_stagec(agg2, cnt, item1, user1, W_r2_ui, W_r2_iu, b2_ui,
                      b2_iu, batch_user.reshape(N, 1), batch_item.reshape(N, 1),
                      lin_W, lin_b)


# speedup vs baseline: 9.6044x; 1.0136x over previous
import functools

import jax
import jax.numpy as jnp
from jax import lax
from jax.experimental import pallas as pl
from jax.experimental.pallas import tpu as pltpu
from jax.experimental.pallas import tpu_sc as plsc

N = 10000
E = 320000
D = 128
H = 64
O = 128
G = 64

NSUB = 16
CH = 128
CHUNKS = -(-E // CH)
CHUNKS_PER_TILE = 160
CHUNKS_PAD = CHUNKS_PER_TILE * NSUB
E_PAD = CHUNKS_PAD * CH
ROWS_PER_TILE = 632
N_PAD = ROWS_PER_TILE * NSUB
ZBLK = 128
TN = N
TROWS = TN // NSUB
IDXB = 32

_mesh = plsc.VectorSubcoreMesh(core_axis_name="c", subcore_axis_name="s")


def _sc_agg_body(with_counts, xw_hbm, src_hbm, dst_hbm, *refs):
    if with_counts:
        (agg_hbm, cnt_hbm, acc_sh, cnt_sh, xw_sh, src_v, dst_v, rows0_v,
         rows1_v, ones_v, zcnt_v, sem0, sem1, sem2) = refs
    else:
        (agg_hbm, acc_sh, xw_sh, src_v, dst_v, rows0_v, rows1_v,
         sem0, sem1) = refs

    cid = lax.axis_index("c")
    sid = lax.axis_index("s")
    zero16 = jnp.zeros((16,), jnp.float32)

    @pl.loop(0, ZBLK)
    def _(r):
        @pl.loop(0, H, step=16)
        def _(k):
            rows0_v[r, pl.ds(k, 16)] = zero16

    base = sid * ROWS_PER_TILE
    @pl.loop(0, 4)
    def _(b):
        pltpu.sync_copy(rows0_v, acc_sh.at[pl.ds(base + b * ZBLK, ZBLK)])
    pltpu.sync_copy(rows0_v.at[pl.ds(0, ROWS_PER_TILE - 4 * ZBLK)],
                    acc_sh.at[pl.ds(base + 4 * ZBLK, ROWS_PER_TILE - 4 * ZBLK)])

    if with_counts:
        one16 = jnp.ones((16,), jnp.float32)

        @pl.loop(0, ZBLK)
        def _(r):
            ones_v[r, pl.ds(0, 16)] = one16
            zcnt_v[r, pl.ds(0, 16)] = zero16

        @pl.loop(0, 4)
        def _(b):
            pltpu.sync_copy(zcnt_v, cnt_sh.at[pl.ds(base + b * ZBLK, ZBLK)])
        pltpu.sync_copy(zcnt_v.at[pl.ds(0, ROWS_PER_TILE - 4 * ZBLK)],
                        cnt_sh.at[pl.ds(base + 4 * ZBLK, ROWS_PER_TILE - 4 * ZBLK)])

    pltpu.sync_copy(xw_hbm.at[pl.ds(cid * TN + sid * TROWS, TROWS)],
                    xw_sh.at[pl.ds(sid * TROWS, TROWS)])
    plsc.subcore_barrier()

    def gstart(j, buf, sem):
        pltpu.async_copy(xw_sh.at[src_v.at[j]], buf, sem)

    def gwait(j, buf, sem):
        pltpu.make_async_copy(xw_sh.at[src_v.at[j]], buf, sem).wait()

    def scat(j, buf):
        pltpu.sync_copy(buf, acc_sh.at[dst_v.at[j]], add=True)
        if with_counts:
            pltpu.async_copy(ones_v, cnt_sh.at[dst_v.at[j]], sem2, add=True)

    def cdrain(j):
        if with_counts:
            pltpu.make_async_copy(ones_v, cnt_sh.at[dst_v.at[j]], sem2).wait()

    tb = sid * CHUNKS_PER_TILE

    @pl.loop(0, CHUNKS_PER_TILE // IDXB)
    def _(b):
        blk = tb + b * IDXB
        pltpu.sync_copy(src_hbm.at[cid, pl.ds(blk, IDXB)], src_v)
        pltpu.sync_copy(dst_hbm.at[cid, pl.ds(blk, IDXB)], dst_v)
        gstart(0, rows0_v, sem0)

        @pl.loop(0, IDXB, step=2)
        def _(j):
            gstart(j + 1, rows1_v, sem1)
            gwait(j, rows0_v, sem0)
            scat(j, rows0_v)

            @pl.when(j >= 2)
            def _():
                cdrain(j - 2)

            @pl.when(j + 2 < IDXB)
            def _():
                gstart(j + 2, rows0_v, sem0)

            gwait(j + 1, rows1_v, sem1)
            scat(j + 1, rows1_v)

            @pl.when(j >= 2)
            def _():
                cdrain(j - 1)

        cdrain(IDXB - 2)
        cdrain(IDXB - 1)

    plsc.subcore_barrier()

    pltpu.sync_copy(acc_sh.at[pl.ds(base, ROWS_PER_TILE)],
                    agg_hbm.at[cid, pl.ds(base, ROWS_PER_TILE)])
    if with_counts:
        pltpu.sync_copy(cnt_sh.at[pl.ds(base, ROWS_PER_TILE)],
                        cnt_hbm.at[cid, pl.ds(base, ROWS_PER_TILE)])


def _make_sc_agg(with_counts):
    out_type = [jax.ShapeDtypeStruct((2, N_PAD, H), jnp.float32)]
    scratch = [
        pltpu.VMEM_SHARED((N_PAD, H), jnp.float32),
    ]
    if with_counts:
        out_type.append(jax.ShapeDtypeStruct((2, N_PAD, 16), jnp.float32))
        scratch.append(pltpu.VMEM_SHARED((N_PAD, 16), jnp.float32))
    scratch += [
        pltpu.VMEM_SHARED((TN, H), jnp.float32),
        pltpu.VMEM((IDXB, CH), jnp.int32),
        pltpu.VMEM((IDXB, CH), jnp.int32),
        pltpu.VMEM((CH, H), jnp.float32),
        pltpu.VMEM((CH, H), jnp.float32),
    ]
    if with_counts:
        scratch += [pltpu.VMEM((CH, 16), jnp.float32),
                    pltpu.VMEM((ZBLK, 16), jnp.float32)]
    scratch += [pltpu.SemaphoreType.DMA, pltpu.SemaphoreType.DMA]
    if with_counts:
        scratch.append(pltpu.SemaphoreType.DMA)
    return pl.kernel(
        functools.partial(_sc_agg_body, with_counts),
        out_type=tuple(out_type) if with_counts else out_type[0],
        mesh=_mesh,
        scratch_types=scratch,
        compiler_params=pltpu.CompilerParams(use_tc_tiling_on_sc=False),
    )


_sc_agg_counts = _make_sc_agg(True)
_sc_agg_plain = _make_sc_agg(False)


def _dotT(x, w):
    return lax.dot_general(x, w, (((1,), (1,)), ((), ())),
                           preferred_element_type=jnp.float32)


def _pre1_body(xu, xi, wui, wiu, out):
    out[pl.ds(0, N), :] = _dotT(xu[...], wui[...])
    out[pl.ds(N, N), :] = _dotT(xi[...], wiu[...])


def _tc_pre1(x_user, x_item, wl_ui, wl_iu):
    return pl.pallas_call(
        _pre1_body,
        out_shape=jax.ShapeDtypeStruct((2 * N, H), jnp.float32),
    )(x_user, x_item, wl_ui, wl_iu)


def _stageb_body(agg, cnt, xu, xi, wr_ui, wr_iu, b_ui, b_iu, wl2_ui, wl2_iu,
                 item1_o, user1_o, xw2_o):
    cnt_ui = jnp.maximum(cnt[0, pl.ds(0, N), pl.ds(0, 1)], 1.0)
    cnt_iu = jnp.maximum(cnt[1, pl.ds(0, N), pl.ds(0, 1)], 1.0)
    item1 = jax.nn.relu(agg[0, pl.ds(0, N), :] / cnt_ui + b_ui[...]
                        + _dotT(xi[...], wr_ui[...]))
    user1 = jax.nn.relu(agg[1, pl.ds(0, N), :] / cnt_iu + b_iu[...]
                        + _dotT(xu[...], wr_iu[...]))
    item1_o[...] = item1
    user1_o[...] = user1
    xw2_o[pl.ds(0, N), :] = _dotT(user1, wl2_ui[...])
    xw2_o[pl.ds(N, N), :] = _dotT(item1, wl2_iu[...])


def _tc_stageb(agg, cnt, x_user, x_item, wr_ui, wr_iu, b_ui, b_iu,
               wl2_ui, wl2_iu):
    return pl.pallas_call(
        _stageb_body,
        out_shape=(
            jax.ShapeDtypeStruct((N, H), jnp.float32),
            jax.ShapeDtypeStruct((N, H), jnp.float32),
            jax.ShapeDtypeStruct((2 * N, H), jnp.float32),
        ),
    )(agg, cnt, x_user, x_item, wr_ui, wr_iu, b_ui, b_iu, wl2_ui, wl2_iu)


def _stagec_body(agg, cnt, item1, user1, wr_ui, wr_iu, b_ui, b_iu,
                 batch_u, batch_i, lin_w, lin_b, out):
    cnt_ui = jnp.maximum(cnt[0, pl.ds(0, N), pl.ds(0, 1)], 1.0)
    cnt_iu = jnp.maximum(cnt[1, pl.ds(0, N), pl.ds(0, 1)], 1.0)
    item2 = jax.nn.relu(agg[0, pl.ds(0, N), :] / cnt_ui + b_ui[...]
                        + _dotT(item1[...], wr_ui[...]))
    user2 = jax.nn.relu(agg[1, pl.ds(0, N), :] / cnt_iu + b_iu[...]
                        + _dotT(user1[...], wr_iu[...]))
    gids = lax.broadcasted_iota(jnp.int32, (1, G), 1)
    oh_u = (batch_u[...] == gids).astype(jnp.float32)
    oh_i = (batch_i[...] == gids).astype(jnp.float32)
    pool_dims = (((0,), (0,)), ((), ()))
    pu = lax.dot_general(oh_u, user2, pool_dims,
                         preferred_element_type=jnp.float32)
    pi = lax.dot_general(oh_i, item2, pool_dims,
                         preferred_element_type=jnp.float32)
    cu = jnp.maximum(jnp.sum(oh_u, axis=0, keepdims=True), 1.0)
    ci = jnp.maximum(jnp.sum(oh_i, axis=0, keepdims=True), 1.0)
    g = pu / cu.T + pi / ci.T
    out[...] = _dotT(g, lin_w[...]) + lin_b[...]


def _tc_stagec(agg, cnt, item1, user1, wr_ui, wr_iu, b_ui, b_iu,
               batch_u, batch_i, lin_w, lin_b):
    return pl.pallas_call(
        _stagec_body,
        out_shape=jax.ShapeDtypeStruct((G, O), jnp.float32),
    )(agg, cnt, item1, user1, wr_ui, wr_iu, b_ui, b_iu,
      batch_u, batch_i, lin_w, lin_b)


def kernel(x_user, x_item, edge_index_ui, edge_index_iu, batch_user,
           batch_item, W_l1_ui, b1_ui, W_r1_ui, W_l1_iu, b1_iu, W_r1_iu,
           W_l2_ui, b2_ui, W_r2_ui, W_l2_iu, b2_iu, W_r2_iu, lin_W, lin_b):
    pad = E_PAD - E
    src_all = jnp.stack([
        jnp.concatenate([edge_index_ui[0], jnp.zeros((pad,), jnp.int32)]),
        jnp.concatenate([edge_index_iu[0], jnp.zeros((pad,), jnp.int32)]),
    ]).reshape(2, CHUNKS_PAD, CH)
    dump = jnp.full((pad,), N, jnp.int32)
    dst_all = jnp.stack([
        jnp.concatenate([edge_index_ui[1], dump]),
        jnp.concatenate([edge_index_iu[1], dump]),
    ]).reshape(2, CHUNKS_PAD, CH)

    xw1 = _tc_pre1(x_user, x_item, W_l1_ui, W_l1_iu)
    agg1, cnt = _sc_agg_counts(xw1, src_all, dst_all)
    item1, user1, xw2 = _tc_stageb(agg1, cnt, x_user, x_item, W_r1_ui,
                                   W_r1_iu, b1_ui, b1_iu, W_l2_ui, W_l2_iu)
    agg2 = _sc_agg_plain(xw2, src_all, dst_all)
    return _tc_stagec(agg2, cnt, item1, user1, W_r2_ui, W_r2_iu, b2_ui,
                      b2_iu, batch_user.reshape(N, 1), batch_item.reshape(N, 1),
                      lin_W, lin_b)


# speedup vs baseline: 10.0001x; 1.0412x over previous
import functools

import jax
import jax.numpy as jnp
from jax import lax
from jax.experimental import pallas as pl
from jax.experimental.pallas import tpu as pltpu
from jax.experimental.pallas import tpu_sc as plsc

N = 10000
E = 320000
D = 128
H = 64
O = 128
G = 64

NSUB = 16
CH = 128
CHUNKS = -(-E // CH)
CHUNKS_PER_TILE = 160
CHUNKS_PAD = CHUNKS_PER_TILE * NSUB
E_PAD = CHUNKS_PAD * CH
ROWS_PER_TILE = 632
N_PAD = ROWS_PER_TILE * NSUB
ZBLK = 128
TN = N
TROWS = TN // NSUB
IDXB = 16
ZCB = 64

_mesh = plsc.VectorSubcoreMesh(core_axis_name="c", subcore_axis_name="s")


def _sc_agg_body(with_counts, xw_hbm, src_hbm, dst_hbm, *refs):
    if with_counts:
        (agg_hbm, cnt_hbm, acc_sh, cnt_sh, xw_sh, src_v, dst_v, rows0_v,
         rows1_v, rows2_v, rows3_v, ones_v, zcnt_v, gs0, gs1, gs2, gs3,
         ss0, ss1, ss2, ss3, csem) = refs
    else:
        (agg_hbm, acc_sh, xw_sh, src_v, dst_v, rows0_v, rows1_v, rows2_v,
         rows3_v, gs0, gs1, gs2, gs3, ss0, ss1, ss2, ss3) = refs
    rows = (rows0_v, rows1_v, rows2_v, rows3_v)
    gsems = (gs0, gs1, gs2, gs3)
    ssems = (ss0, ss1, ss2, ss3)

    cid = lax.axis_index("c")
    sid = lax.axis_index("s")
    zero16 = jnp.zeros((16,), jnp.float32)

    @pl.loop(0, ZBLK)
    def _(r):
        @pl.loop(0, H, step=16)
        def _(k):
            rows0_v[r, pl.ds(k, 16)] = zero16

    base = sid * ROWS_PER_TILE
    @pl.loop(0, 4)
    def _(b):
        pltpu.sync_copy(rows0_v, acc_sh.at[pl.ds(base + b * ZBLK, ZBLK)])
    pltpu.sync_copy(rows0_v.at[pl.ds(0, ROWS_PER_TILE - 4 * ZBLK)],
                    acc_sh.at[pl.ds(base + 4 * ZBLK, ROWS_PER_TILE - 4 * ZBLK)])

    if with_counts:
        one16 = jnp.ones((16,), jnp.float32)

        @pl.loop(0, ZBLK)
        def _(r):
            ones_v[r, pl.ds(0, 16)] = one16

        @pl.loop(0, ZCB)
        def _(r):
            zcnt_v[r, pl.ds(0, 16)] = zero16

        @pl.loop(0, 9)
        def _(b):
            pltpu.sync_copy(zcnt_v, cnt_sh.at[pl.ds(base + b * ZCB, ZCB)])
        pltpu.sync_copy(zcnt_v.at[pl.ds(0, ROWS_PER_TILE - 9 * ZCB)],
                        cnt_sh.at[pl.ds(base + 9 * ZCB, ROWS_PER_TILE - 9 * ZCB)])

    pltpu.sync_copy(xw_hbm.at[pl.ds(cid * TN + sid * TROWS, TROWS)],
                    xw_sh.at[pl.ds(sid * TROWS, TROWS)])
    plsc.subcore_barrier()

    def gstart(j, k):
        pltpu.async_copy(xw_sh.at[src_v.at[j]], rows[k], gsems[k])

    def gwait(j, k):
        pltpu.make_async_copy(xw_sh.at[src_v.at[j]], rows[k], gsems[k]).wait()

    def sstart(j, k):
        pltpu.async_copy(rows[k], acc_sh.at[dst_v.at[j]], ssems[k], add=True)
        if with_counts:
            pltpu.async_copy(ones_v, cnt_sh.at[dst_v.at[j]], csem, add=True)

    def swait(j, k):
        pltpu.make_async_copy(rows[k], acc_sh.at[dst_v.at[j]], ssems[k]).wait()

    def cdrain(j):
        if with_counts:
            pltpu.make_async_copy(ones_v, cnt_sh.at[dst_v.at[j]], csem).wait()

    tb = sid * CHUNKS_PER_TILE

    @pl.loop(0, CHUNKS_PER_TILE // IDXB)
    def _(b):
        blk = tb + b * IDXB
        pltpu.sync_copy(src_hbm.at[cid, pl.ds(blk, IDXB)], src_v)
        pltpu.sync_copy(dst_hbm.at[cid, pl.ds(blk, IDXB)], dst_v)
        gstart(0, 0)
        gstart(1, 1)
        gstart(2, 2)

        @pl.loop(0, IDXB, step=4)
        def _(c):
            for k in range(4):
                cc = c + k
                gwait(cc, k)
                sstart(cc, k)
                nxt_k = (k + 3) % 4

                @pl.when(cc + 3 < IDXB)
                def _():
                    @pl.when(cc >= 1)
                    def _():
                        swait(cc - 1, nxt_k)
                    gstart(cc + 3, nxt_k)

                @pl.when(cc >= 2)
                def _():
                    cdrain(cc - 2)

        swait(IDXB - 4, (IDXB - 4) % 4)
        swait(IDXB - 3, (IDXB - 3) % 4)
        swait(IDXB - 2, (IDXB - 2) % 4)
        swait(IDXB - 1, (IDXB - 1) % 4)
        cdrain(IDXB - 2)
        cdrain(IDXB - 1)

    plsc.subcore_barrier()

    pltpu.sync_copy(acc_sh.at[pl.ds(base, ROWS_PER_TILE)],
                    agg_hbm.at[cid, pl.ds(base, ROWS_PER_TILE)])
    if with_counts:
        pltpu.sync_copy(cnt_sh.at[pl.ds(base, ROWS_PER_TILE)],
                        cnt_hbm.at[cid, pl.ds(base, ROWS_PER_TILE)])


def _make_sc_agg(with_counts):
    out_type = [jax.ShapeDtypeStruct((2, N_PAD, H), jnp.float32)]
    scratch = [
        pltpu.VMEM_SHARED((N_PAD, H), jnp.float32),
    ]
    if with_counts:
        out_type.append(jax.ShapeDtypeStruct((2, N_PAD, 16), jnp.float32))
        scratch.append(pltpu.VMEM_SHARED((N_PAD, 16), jnp.float32))
    scratch += [
        pltpu.VMEM_SHARED((TN, H), jnp.float32),
        pltpu.VMEM((IDXB, CH), jnp.int32),
        pltpu.VMEM((IDXB, CH), jnp.int32),
        pltpu.VMEM((CH, H), jnp.float32),
        pltpu.VMEM((CH, H), jnp.float32),
        pltpu.VMEM((CH, H), jnp.float32),
        pltpu.VMEM((CH, H), jnp.float32),
    ]
    if with_counts:
        scratch += [pltpu.VMEM((CH, 16), jnp.float32),
                    pltpu.VMEM((ZCB, 16), jnp.float32)]
    scratch += [pltpu.SemaphoreType.DMA] * 8
    if with_counts:
        scratch.append(pltpu.SemaphoreType.DMA)
    return pl.kernel(
        functools.partial(_sc_agg_body, with_counts),
        out_type=tuple(out_type) if with_counts else out_type[0],
        mesh=_mesh,
        scratch_types=scratch,
        compiler_params=pltpu.CompilerParams(use_tc_tiling_on_sc=False),
    )


_sc_agg_counts = _make_sc_agg(True)
_sc_agg_plain = _make_sc_agg(False)


def _dotT(x, w):
    return lax.dot_general(x, w, (((1,), (1,)), ((), ())),
                           preferred_element_type=jnp.float32)


def _pre1_body(xu, xi, wui, wiu, out):
    out[pl.ds(0, N), :] = _dotT(xu[...], wui[...])
    out[pl.ds(N, N), :] = _dotT(xi[...], wiu[...])


def _tc_pre1(x_user, x_item, wl_ui, wl_iu):
    return pl.pallas_call(
        _pre1_body,
        out_shape=jax.ShapeDtypeStruct((2 * N, H), jnp.float32),
    )(x_user, x_item, wl_ui, wl_iu)


def _stageb_body(agg, cnt, xu, xi, wr_ui, wr_iu, b_ui, b_iu, wl2_ui, wl2_iu,
                 item1_o, user1_o, xw2_o):
    cnt_ui = jnp.maximum(cnt[0, pl.ds(0, N), pl.ds(0, 1)], 1.0)
    cnt_iu = jnp.maximum(cnt[1, pl.ds(0, N), pl.ds(0, 1)], 1.0)
    item1 = jax.nn.relu(agg[0, pl.ds(0, N), :] / cnt_ui + b_ui[...]
                        + _dotT(xi[...], wr_ui[...]))
    user1 = jax.nn.relu(agg[1, pl.ds(0, N), :] / cnt_iu + b_iu[...]
                        + _dotT(xu[...], wr_iu[...]))
    item1_o[...] = item1
    user1_o[...] = user1
    xw2_o[pl.ds(0, N), :] = _dotT(user1, wl2_ui[...])
    xw2_o[pl.ds(N, N), :] = _dotT(item1, wl2_iu[...])


def _tc_stageb(agg, cnt, x_user, x_item, wr_ui, wr_iu, b_ui, b_iu,
               wl2_ui, wl2_iu):
    return pl.pallas_call(
        _stageb_body,
        out_shape=(
            jax.ShapeDtypeStruct((N, H), jnp.float32),
            jax.ShapeDtypeStruct((N, H), jnp.float32),
            jax.ShapeDtypeStruct((2 * N, H), jnp.float32),
        ),
    )(agg, cnt, x_user, x_item, wr_ui, wr_iu, b_ui, b_iu, wl2_ui, wl2_iu)


def _stagec_body(agg, cnt, item1, user1, wr_ui, wr_iu, b_ui, b_iu,
                 batch_u, batch_i, lin_w, lin_b, out):
    cnt_ui = jnp.maximum(cnt[0, pl.ds(0, N), pl.ds(0, 1)], 1.0)
    cnt_iu = jnp.maximum(cnt[1, pl.ds(0, N), pl.ds(0, 1)], 1.0)
    item2 = jax.nn.relu(agg[0, pl.ds(0, N), :] / cnt_ui + b_ui[...]
                        + _dotT(item1[...], wr_ui[...]))
    user2 = jax.nn.relu(agg[1, pl.ds(0, N), :] / cnt_iu + b_iu[...]
                        + _dotT(user1[...], wr_iu[...]))
    gids = lax.broadcasted_iota(jnp.int32, (1, G), 1)
    oh_u = (batch_u[...] == gids).astype(jnp.float32)
    oh_i = (batch_i[...] == gids).astype(jnp.float32)
    pool_dims = (((0,), (0,)), ((), ()))
    pu = lax.dot_general(oh_u, user2, pool_dims,
                         preferred_element_type=jnp.float32)
    pi = lax.dot_general(oh_i, item2, pool_dims,
                         preferred_element_type=jnp.float32)
    cu = jnp.maximum(jnp.sum(oh_u, axis=0, keepdims=True), 1.0)
    ci = jnp.maximum(jnp.sum(oh_i, axis=0, keepdims=True), 1.0)
    g = pu / cu.T + pi / ci.T
    out[...] = _dotT(g, lin_w[...]) + lin_b[...]


def _tc_stagec(agg, cnt, item1, user1, wr_ui, wr_iu, b_ui, b_iu,
               batch_u, batch_i, lin_w, lin_b):
    return pl.pallas_call(
        _stagec_body,
        out_shape=jax.ShapeDtypeStruct((G, O), jnp.float32),
    )(agg, cnt, item1, user1, wr_ui, wr_iu, b_ui, b_iu,
      batch_u, batch_i, lin_w, lin_b)


def kernel(x_user, x_item, edge_index_ui, edge_index_iu, batch_user,
           batch_item, W_l1_ui, b1_ui, W_r1_ui, W_l1_iu, b1_iu, W_r1_iu,
           W_l2_ui, b2_ui, W_r2_ui, W_l2_iu, b2_iu, W_r2_iu, lin_W, lin_b):
    pad = E_PAD - E
    src_all = jnp.stack([
        jnp.concatenate([edge_index_ui[0], jnp.zeros((pad,), jnp.int32)]),
        jnp.concatenate([edge_index_iu[0], jnp.zeros((pad,), jnp.int32)]),
    ]).reshape(2, CHUNKS_PAD, CH)
    dump = jnp.full((pad,), N, jnp.int32)
    dst_all = jnp.stack([
        jnp.concatenate([edge_index_ui[1], dump]),
        jnp.concatenate([edge_index_iu[1], dump]),
    ]).reshape(2, CHUNKS_PAD, CH)

    xw1 = _tc_pre1(x_user, x_item, W_l1_ui, W_l1_iu)
    agg1, cnt = _sc_agg_counts(xw1, src_all, dst_all)
    item1, user1, xw2 = _tc_stageb(agg1, cnt, x_user, x_item, W_r1_ui,
                                   W_r1_iu, b1_ui, b1_iu, W_l2_ui, W_l2_iu)
    agg2 = _sc_agg_plain(xw2, src_all, dst_all)
    return _tc_stagec(agg2, cnt, item1, user1, W_r2_ui, W_r2_iu, b2_ui,
                      b2_iu, batch_user.reshape(N, 1), batch_item.reshape(N, 1),
                      lin_W, lin_b)


# speedup vs baseline: 10.0360x; 1.0036x over previous
import functools

import jax
import jax.numpy as jnp
from jax import lax
from jax.experimental import pallas as pl
from jax.experimental.pallas import tpu as pltpu
from jax.experimental.pallas import tpu_sc as plsc

N = 10000
E = 320000
D = 128
H = 64
O = 128
G = 64

NSUB = 16
CH = 128
CHUNKS = E // CH
BASE_CPT = CHUNKS // NSUB
EXTRA = CHUNKS - BASE_CPT * NSUB
ROWS_PER_TILE = 632
N_PAD = ROWS_PER_TILE * NSUB
ZBLK = 128
TN = N
TROWS = TN // NSUB
IDXB = 12
NBLKS = BASE_CPT // IDXB
ZCB = 64

_mesh = plsc.VectorSubcoreMesh(core_axis_name="c", subcore_axis_name="s")


def _sc_agg_body(with_counts, xw_hbm, src_ui_hbm, dst_ui_hbm, src_iu_hbm,
                 dst_iu_hbm, *refs):
    if with_counts:
        (agg_hbm, cnt_hbm, acc_sh, cnt_sh, xw_sh, src_v, dst_v, rows0_v,
         rows1_v, rows2_v, rows3_v, ones_v, zcnt_v, gs0, gs1, gs2, gs3,
         ss0, ss1, ss2, ss3, csem) = refs
    else:
        (agg_hbm, acc_sh, xw_sh, src_v, dst_v, rows0_v, rows1_v, rows2_v,
         rows3_v, gs0, gs1, gs2, gs3, ss0, ss1, ss2, ss3) = refs
    rows = (rows0_v, rows1_v, rows2_v, rows3_v)
    gsems = (gs0, gs1, gs2, gs3)
    ssems = (ss0, ss1, ss2, ss3)

    cid = lax.axis_index("c")
    sid = lax.axis_index("s")
    zero16 = jnp.zeros((16,), jnp.float32)

    @pl.loop(0, ZBLK)
    def _(r):
        @pl.loop(0, H, step=16)
        def _(k):
            rows0_v[r, pl.ds(k, 16)] = zero16

    base = sid * ROWS_PER_TILE
    @pl.loop(0, 4)
    def _(b):
        pltpu.sync_copy(rows0_v, acc_sh.at[pl.ds(base + b * ZBLK, ZBLK)])
    pltpu.sync_copy(rows0_v.at[pl.ds(0, ROWS_PER_TILE - 4 * ZBLK)],
                    acc_sh.at[pl.ds(base + 4 * ZBLK, ROWS_PER_TILE - 4 * ZBLK)])

    if with_counts:
        one16 = jnp.ones((16,), jnp.float32)

        @pl.loop(0, ZBLK)
        def _(r):
            ones_v[r, pl.ds(0, 16)] = one16

        @pl.loop(0, ZCB)
        def _(r):
            zcnt_v[r, pl.ds(0, 16)] = zero16

        @pl.loop(0, 9)
        def _(b):
            pltpu.sync_copy(zcnt_v, cnt_sh.at[pl.ds(base + b * ZCB, ZCB)])
        pltpu.sync_copy(zcnt_v.at[pl.ds(0, ROWS_PER_TILE - 9 * ZCB)],
                        cnt_sh.at[pl.ds(base + 9 * ZCB, ROWS_PER_TILE - 9 * ZCB)])

    pltpu.sync_copy(xw_hbm.at[pl.ds(cid * TN + sid * TROWS, TROWS)],
                    xw_sh.at[pl.ds(sid * TROWS, TROWS)])
    plsc.subcore_barrier()

    def gstart(j, k):
        pltpu.async_copy(xw_sh.at[src_v.at[j]], rows[k], gsems[k])

    def gwait(j, k):
        pltpu.make_async_copy(xw_sh.at[src_v.at[j]], rows[k], gsems[k]).wait()

    def sstart(j, k):
        pltpu.async_copy(rows[k], acc_sh.at[dst_v.at[j]], ssems[k], add=True)
        if with_counts:
            pltpu.async_copy(ones_v, cnt_sh.at[dst_v.at[j]], csem, add=True)

    def swait(j, k):
        pltpu.make_async_copy(rows[k], acc_sh.at[dst_v.at[j]], ssems[k]).wait()

    def cdrain(j):
        if with_counts:
            pltpu.make_async_copy(ones_v, cnt_sh.at[dst_v.at[j]], csem).wait()

    def run_edges(src_hbm, dst_hbm):
        start = BASE_CPT * sid + jnp.minimum(sid, EXTRA)

        @pl.loop(0, NBLKS)
        def _(b):
            blk = start + b * IDXB
            pltpu.sync_copy(src_hbm.at[pl.ds(blk, IDXB)], src_v)
            pltpu.sync_copy(dst_hbm.at[pl.ds(blk, IDXB)], dst_v)
            gstart(0, 0)
            gstart(1, 1)
            gstart(2, 2)

            @pl.loop(0, IDXB, step=4)
            def _(c):
                for k in range(4):
                    cc = c + k
                    gwait(cc, k)
                    sstart(cc, k)
                    nxt_k = (k + 3) % 4

                    @pl.when(cc + 3 < IDXB)
                    def _():
                        @pl.when(cc >= 1)
                        def _():
                            swait(cc - 1, nxt_k)
                        gstart(cc + 3, nxt_k)

                    @pl.when(cc >= 2)
                    def _():
                        cdrain(cc - 2)

            swait(IDXB - 4, (IDXB - 4) % 4)
            swait(IDXB - 3, (IDXB - 3) % 4)
            swait(IDXB - 2, (IDXB - 2) % 4)
            swait(IDXB - 1, (IDXB - 1) % 4)
            cdrain(IDXB - 2)
            cdrain(IDXB - 1)

        @pl.when(sid < EXTRA)
        def _():
            e = start + BASE_CPT
            pltpu.sync_copy(src_hbm.at[pl.ds(e, 1)], src_v.at[pl.ds(0, 1)])
            pltpu.sync_copy(dst_hbm.at[pl.ds(e, 1)], dst_v.at[pl.ds(0, 1)])
            gstart(0, 0)
            gwait(0, 0)
            pltpu.sync_copy(rows0_v, acc_sh.at[dst_v.at[0]], add=True)
            if with_counts:
                pltpu.sync_copy(ones_v, cnt_sh.at[dst_v.at[0]], add=True)

    @pl.when(cid == 0)
    def _():
        run_edges(src_ui_hbm, dst_ui_hbm)

    @pl.when(cid == 1)
    def _():
        run_edges(src_iu_hbm, dst_iu_hbm)

    plsc.subcore_barrier()

    pltpu.sync_copy(acc_sh.at[pl.ds(base, ROWS_PER_TILE)],
                    agg_hbm.at[cid, pl.ds(base, ROWS_PER_TILE)])
    if with_counts:
        pltpu.sync_copy(cnt_sh.at[pl.ds(base, ROWS_PER_TILE)],
                        cnt_hbm.at[cid, pl.ds(base, ROWS_PER_TILE)])


def _make_sc_agg(with_counts):
    out_type = [jax.ShapeDtypeStruct((2, N_PAD, H), jnp.float32)]
    scratch = [
        pltpu.VMEM_SHARED((N_PAD, H), jnp.float32),
    ]
    if with_counts:
        out_type.append(jax.ShapeDtypeStruct((2, N_PAD, 16), jnp.float32))
        scratch.append(pltpu.VMEM_SHARED((N_PAD, 16), jnp.float32))
    scratch += [
        pltpu.VMEM_SHARED((TN, H), jnp.float32),
        pltpu.VMEM((IDXB, CH), jnp.int32),
        pltpu.VMEM((IDXB, CH), jnp.int32),
        pltpu.VMEM((CH, H), jnp.float32),
        pltpu.VMEM((CH, H), jnp.float32),
        pltpu.VMEM((CH, H), jnp.float32),
        pltpu.VMEM((CH, H), jnp.float32),
    ]
    if with_counts:
        scratch += [pltpu.VMEM((CH, 16), jnp.float32),
                    pltpu.VMEM((ZCB, 16), jnp.float32)]
    scratch += [pltpu.SemaphoreType.DMA] * 8
    if with_counts:
        scratch.append(pltpu.SemaphoreType.DMA)
    return pl.kernel(
        functools.partial(_sc_agg_body, with_counts),
        out_type=tuple(out_type) if with_counts else out_type[0],
        mesh=_mesh,
        scratch_types=scratch,
        compiler_params=pltpu.CompilerParams(use_tc_tiling_on_sc=False),
    )


_sc_agg_counts = _make_sc_agg(True)
_sc_agg_plain = _make_sc_agg(False)


def _dotT(x, w):
    return lax.dot_general(x, w, (((1,), (1,)), ((), ())),
                           preferred_element_type=jnp.float32)


def _pre1_body(xu, xi, wui, wiu, out):
    out[pl.ds(0, N), :] = _dotT(xu[...], wui[...])
    out[pl.ds(N, N), :] = _dotT(xi[...], wiu[...])


def _tc_pre1(x_user, x_item, wl_ui, wl_iu):
    return pl.pallas_call(
        _pre1_body,
        out_shape=jax.ShapeDtypeStruct((2 * N, H), jnp.float32),
    )(x_user, x_item, wl_ui, wl_iu)


def _stageb_body(agg, cnt, xu, xi, wr_ui, wr_iu, b_ui, b_iu, wl2_ui, wl2_iu,
                 item1_o, user1_o, xw2_o):
    cnt_ui = jnp.maximum(cnt[0, pl.ds(0, N), pl.ds(0, 1)], 1.0)
    cnt_iu = jnp.maximum(cnt[1, pl.ds(0, N), pl.ds(0, 1)], 1.0)
    item1 = jax.nn.relu(agg[0, pl.ds(0, N), :] / cnt_ui + b_ui[...]
                        + _dotT(xi[...], wr_ui[...]))
    user1 = jax.nn.relu(agg[1, pl.ds(0, N), :] / cnt_iu + b_iu[...]
                        + _dotT(xu[...], wr_iu[...]))
    item1_o[...] = item1
    user1_o[...] = user1
    xw2_o[pl.ds(0, N), :] = _dotT(user1, wl2_ui[...])
    xw2_o[pl.ds(N, N), :] = _dotT(item1, wl2_iu[...])


def _tc_stageb(agg, cnt, x_user, x_item, wr_ui, wr_iu, b_ui, b_iu,
               wl2_ui, wl2_iu):
    return pl.pallas_call(
        _stageb_body,
        out_shape=(
            jax.ShapeDtypeStruct((N, H), jnp.float32),
            jax.ShapeDtypeStruct((N, H), jnp.float32),
            jax.ShapeDtypeStruct((2 * N, H), jnp.float32),
        ),
    )(agg, cnt, x_user, x_item, wr_ui, wr_iu, b_ui, b_iu, wl2_ui, wl2_iu)


def _stagec_body(agg, cnt, item1, user1, wr_ui, wr_iu, b_ui, b_iu,
                 batch_u, batch_i, lin_w, lin_b, out):
    cnt_ui = jnp.maximum(cnt[0, pl.ds(0, N), pl.ds(0, 1)], 1.0)
    cnt_iu = jnp.maximum(cnt[1, pl.ds(0, N), pl.ds(0, 1)], 1.0)
    item2 = jax.nn.relu(agg[0, pl.ds(0, N), :] / cnt_ui + b_ui[...]
                        + _dotT(item1[...], wr_ui[...]))
    user2 = jax.nn.relu(agg[1, pl.ds(0, N), :] / cnt_iu + b_iu[...]
                        + _dotT(user1[...], wr_iu[...]))
    gids = lax.broadcasted_iota(jnp.int32, (1, G), 1)
    oh_u = (batch_u[...] == gids).astype(jnp.float32)
    oh_i = (batch_i[...] == gids).astype(jnp.float32)
    pool_dims = (((0,), (0,)), ((), ()))
    pu = lax.dot_general(oh_u, user2, pool_dims,
                         preferred_element_type=jnp.float32)
    pi = lax.dot_general(oh_i, item2, pool_dims,
                         preferred_element_type=jnp.float32)
    cu = jnp.maximum(jnp.sum(oh_u, axis=0, keepdims=True), 1.0)
    ci = jnp.maximum(jnp.sum(oh_i, axis=0, keepdims=True), 1.0)
    g = pu / cu.T + pi / ci.T
    out[...] = _dotT(g, lin_w[...]) + lin_b[...]


def _tc_stagec(agg, cnt, item1, user1, wr_ui, wr_iu, b_ui, b_iu,
               batch_u, batch_i, lin_w, lin_b):
    return pl.pallas_call(
        _stagec_body,
        out_shape=jax.ShapeDtypeStruct((G, O), jnp.float32),
    )(agg, cnt, item1, user1, wr_ui, wr_iu, b_ui, b_iu,
      batch_u, batch_i, lin_w, lin_b)


def kernel(x_user, x_item, edge_index_ui, edge_index_iu, batch_user,
           batch_item, W_l1_ui, b1_ui, W_r1_ui, W_l1_iu, b1_iu, W_r1_iu,
           W_l2_ui, b2_ui, W_r2_ui, W_l2_iu, b2_iu, W_r2_iu, lin_W, lin_b):
    src_ui = edge_index_ui[0].reshape(CHUNKS, CH)
    dst_ui = edge_index_ui[1].reshape(CHUNKS, CH)
    src_iu = edge_index_iu[0].reshape(CHUNKS, CH)
    dst_iu = edge_index_iu[1].reshape(CHUNKS, CH)

    xw1 = _tc_pre1(x_user, x_item, W_l1_ui, W_l1_iu)
    agg1, cnt = _sc_agg_counts(xw1, src_ui, dst_ui, src_iu, dst_iu)
    item1, user1, xw2 = _tc_stageb(agg1, cnt, x_user, x_item, W_r1_ui,
                                   W_r1_iu, b1_ui, b1_iu, W_l2_ui, W_l2_iu)
    agg2 = _sc_agg_plain(xw2, src_ui, dst_ui, src_iu, dst_iu)
    return _tc_stagec(agg2, cnt, item1, user1, W_r2_ui, W_r2_iu, b2_ui,
                      b2_iu, batch_user.reshape(N, 1), batch_item.reshape(N, 1),
                      lin_W, lin_b)
